# Initial kernel scaffold; baseline (speedup 1.0000x reference)
import functools

import jax
import jax.numpy as jnp
from jax import lax
from jax.experimental import pallas as pl
from jax.experimental.pallas import tpu as pltpu
from jax.experimental.pallas import tpu_sc as plsc

N = 10000
E = 160000
D_IN = 256
DH = 128
NS = 16
EPT = E // NS
CH = 80
NCHUNK = EPT // CH
RPT = N // NS

_mesh = plsc.VectorSubcoreMesh(core_axis_name="c", subcore_axis_name="s")


@functools.partial(
    pl.kernel,
    out_type=[
        jax.ShapeDtypeStruct((N, DH), jnp.float32),
        jax.ShapeDtypeStruct((N, DH), jnp.float32),
        jax.ShapeDtypeStruct((N, 16), jnp.float32),
    ],
    mesh=_mesh,
    scratch_types=[
        pltpu.VMEM_SHARED((N, DH), jnp.float32),
        pltpu.VMEM_SHARED((N, 16), jnp.float32),
        pltpu.VMEM((EPT,), jnp.int32),
        pltpu.VMEM((EPT,), jnp.int32),
        pltpu.VMEM((CH,), jnp.int32),
        pltpu.VMEM((CH,), jnp.int32),
        pltpu.VMEM((CH,), jnp.int32),
        pltpu.VMEM((CH,), jnp.int32),
        pltpu.VMEM((CH, DH), jnp.float32),
        pltpu.VMEM((CH, DH), jnp.float32),
        pltpu.VMEM((CH, 16), jnp.float32),
        pltpu.SemaphoreType.DMA,
        pltpu.SemaphoreType.DMA,
    ],
)
def _sc_agg(xs_hbm, src_hbm, dst_hbm, z128_hbm, z16_hbm, o16_hbm,
            out_lo, out_hi, out_deg,
            acc, dacc, srcb, dstb, sidx0, sidx1, didx0, didx1,
            rows0, rows1, ones16, sem0, sem1):
    cid = lax.axis_index("c")
    sid = lax.axis_index("s")

    sidx = (sidx0, sidx1)
    didx = (didx0, didx1)
    rows = (rows0, rows1)
    sems = (sem0, sem1)

    pltpu.sync_copy(z128_hbm, acc.at[pl.ds(sid * RPT, RPT)])
    pltpu.sync_copy(z16_hbm, dacc.at[pl.ds(sid * RPT, RPT)])
    pltpu.sync_copy(o16_hbm, ones16)

    base = cid * E + sid * EPT
    pltpu.sync_copy(src_hbm.at[pl.ds(base, EPT)], srcb)
    pltpu.sync_copy(dst_hbm.at[pl.ds(sid * EPT, EPT)], dstb)

    plsc.subcore_barrier()

    def fill(c, b):
        pltpu.sync_copy(srcb.at[pl.ds(c * CH, CH)], sidx[b])
        pltpu.sync_copy(dstb.at[pl.ds(c * CH, CH)], didx[b])

    def gstart(b):
        return pltpu.async_copy(xs_hbm.at[sidx[b]], rows[b], sems[b])

    def gwait(b):
        pltpu.make_async_copy(xs_hbm.at[pl.ds(0, CH)], rows[b], sems[b]).wait()

    def scatter(b):
        pltpu.sync_copy(rows[b], acc.at[didx[b]], add=True)

        @pl.when(cid == 0)
        def _():
            pltpu.sync_copy(ones16, dacc.at[didx[b]], add=True)

    fill(0, 0)
    gstart(0)

    def body(j, carry):
        c = 2 * j
        fill(c + 1, 1)
        gstart(1)
        gwait(0)
        scatter(0)
        fill(c + 2, 0)
        gstart(0)
        gwait(1)
        scatter(1)
        return carry

    lax.fori_loop(0, (NCHUNK - 1) // 2, body, 0)
    gwait(0)
    scatter(0)

    plsc.subcore_barrier()

    @pl.when(cid == 0)
    def _():
        pltpu.sync_copy(acc.at[pl.ds(sid * RPT, RPT)],
                        out_lo.at[pl.ds(sid * RPT, RPT)])
        pltpu.sync_copy(dacc.at[pl.ds(sid * RPT, RPT)],
                        out_deg.at[pl.ds(sid * RPT, RPT)])

    @pl.when(cid == 1)
    def _():
        pltpu.sync_copy(acc.at[pl.ds(sid * RPT, RPT)],
                        out_hi.at[pl.ds(sid * RPT, RPT)])


ROWS_BLK = 2000


def _tc_body(aglo_ref, aghi_ref, x_ref, deg_ref, w_ref, b_ref, o_ref):
    r = 1.0 / (deg_ref[:, 0:1] + 1.0)
    hlo = (aglo_ref[...] + x_ref[:, :DH]) * r
    hhi = (aghi_ref[...] + x_ref[:, DH:]) * r
    o_ref[...] = (
        jnp.dot(hlo, w_ref[:DH, :], preferred_element_type=jnp.float32)
        + jnp.dot(hhi, w_ref[DH:, :], preferred_element_type=jnp.float32)
        + b_ref[...]
    )


def _tc_call(agg_lo, agg_hi, x, deg_rows, W, b2):
    grid = (N // ROWS_BLK,)
    return pl.pallas_call(
        _tc_body,
        grid=grid,
        in_specs=[
            pl.BlockSpec((ROWS_BLK, DH), lambda i: (i, 0)),
            pl.BlockSpec((ROWS_BLK, DH), lambda i: (i, 0)),
            pl.BlockSpec((ROWS_BLK, D_IN), lambda i: (i, 0)),
            pl.BlockSpec((ROWS_BLK, 16), lambda i: (i, 0)),
            pl.BlockSpec((D_IN, D_IN), lambda i: (0, 0)),
            pl.BlockSpec((1, D_IN), lambda i: (0, 0)),
        ],
        out_specs=pl.BlockSpec((ROWS_BLK, D_IN), lambda i: (i, 0)),
        out_shape=jax.ShapeDtypeStruct((N, D_IN), jnp.float32),
    )(agg_lo, agg_hi, x, deg_rows, W, b2)


@jax.jit
def kernel(x, edge_index, W, b):
    src = edge_index[0].astype(jnp.int32)
    dst = edge_index[1].astype(jnp.int32)
    xs = jnp.concatenate([x[:, :DH], x[:, DH:]], axis=0)
    src2 = jnp.concatenate([src, src + N])
    z128 = jnp.zeros((RPT, DH), jnp.float32)
    z16 = jnp.zeros((RPT, 16), jnp.float32)
    o16 = jnp.ones((CH, 16), jnp.float32)
    agg_lo, agg_hi, deg_rows = _sc_agg(xs, src2, dst, z128, z16, o16)
    return _tc_call(agg_lo, agg_hi, x, deg_rows, W, b.reshape(1, D_IN))


# speedup vs baseline: 7.7233x; 7.7233x over previous
import functools

import jax
import jax.numpy as jnp
from jax import lax
from jax.experimental import pallas as pl
from jax.experimental.pallas import tpu as pltpu
from jax.experimental.pallas import tpu_sc as plsc

N = 10000
E = 160000
D_IN = 256
DH = 128
NS = 16
EPT = E // NS
CH = 80
NCHUNK = EPT // CH
ZR = 624
ZR_LAST = N - 15 * ZR

_mesh = plsc.VectorSubcoreMesh(core_axis_name="c", subcore_axis_name="s")


@functools.partial(
    pl.kernel,
    out_type=[
        jax.ShapeDtypeStruct((N, DH), jnp.float32),
        jax.ShapeDtypeStruct((N, DH), jnp.float32),
        jax.ShapeDtypeStruct((N,), jnp.float32),
        jax.ShapeDtypeStruct((N,), jnp.float32),
    ],
    mesh=_mesh,
    scratch_types=[
        pltpu.VMEM_SHARED((N, DH), jnp.float32),
        pltpu.VMEM_SHARED((N,), jnp.float32),
        pltpu.VMEM((CH,), jnp.int32),
        pltpu.VMEM((CH,), jnp.int32),
        pltpu.VMEM((CH,), jnp.int32),
        pltpu.VMEM((CH,), jnp.int32),
        pltpu.VMEM((CH, DH), jnp.float32),
        pltpu.VMEM((CH, DH), jnp.float32),
        pltpu.VMEM((CH,), jnp.float32),
        pltpu.VMEM((ZR_LAST,), jnp.float32),
        pltpu.SemaphoreType.DMA,
        pltpu.SemaphoreType.DMA,
        pltpu.SemaphoreType.DMA,
        pltpu.SemaphoreType.DMA,
        pltpu.SemaphoreType.DMA,
        pltpu.SemaphoreType.DMA,
        pltpu.SemaphoreType.DMA,
    ],
)
def _sc_agg(xs_hbm, src_hbm, dst_hbm, z128_hbm, z1_hbm,
            out_lo, out_hi, out_deg0, out_deg1,
            acc, dacc, sidx0, sidx1, didx0, didx1,
            rows0, rows1, dones, dbuf, sem0, sem1,
            ssm0, ssm1, dsm0, dsm1, degsem):
    cid = lax.axis_index("c")
    sid = lax.axis_index("s")

    sidx = (sidx0, sidx1)
    didx = (didx0, didx1)
    rows = (rows0, rows1)
    sems = (sem0, sem1)
    ssms = (ssm0, ssm1)
    dsms = (dsm0, dsm1)

    pltpu.sync_copy(z1_hbm, dbuf)

    @pl.when(sid < 15)
    def _():
        st = pl.multiple_of(sid * ZR, 8)
        pltpu.sync_copy(z128_hbm.at[pl.ds(0, ZR)], acc.at[pl.ds(st, ZR)])
        pltpu.sync_copy(dbuf.at[pl.ds(0, ZR)], dacc.at[pl.ds(st, ZR)])

    @pl.when(sid == 15)
    def _():
        pltpu.sync_copy(z128_hbm, acc.at[pl.ds(15 * ZR, ZR_LAST)])
        pltpu.sync_copy(dbuf, dacc.at[pl.ds(15 * ZR, ZR_LAST)])

    for i in range(CH // 16):
        dones[pl.ds(i * 16, 16)] = jnp.full((16,), 1.0, jnp.float32)

    base = cid * E + sid * EPT

    def sfill_start(c, b):
        pltpu.async_copy(src_hbm.at[pl.ds(base + c * CH, CH)], sidx[b], ssms[b])

    def sfill_wait(b):
        pltpu.make_async_copy(src_hbm.at[pl.ds(0, CH)], sidx[b], ssms[b]).wait()

    def dfill_start(c, b):
        pltpu.async_copy(dst_hbm.at[pl.ds(sid * EPT + c * CH, CH)], didx[b],
                         dsms[b])

    def dfill_wait(b):
        pltpu.make_async_copy(dst_hbm.at[pl.ds(0, CH)], didx[b], dsms[b]).wait()

    def gstart(b):
        return pltpu.async_copy(xs_hbm.at[sidx[b]], rows[b], sems[b])

    def gwait(b):
        pltpu.make_async_copy(xs_hbm.at[pl.ds(0, CH)], rows[b], sems[b]).wait()

    def scatter(b):
        dfill_wait(b)

        @pl.when(cid == b)
        def _():
            pltpu.async_copy(dones, dacc.at[didx[b]], degsem, add=True)

        pltpu.sync_copy(rows[b], acc.at[didx[b]], add=True)

        @pl.when(cid == b)
        def _():
            pltpu.make_async_copy(dones, dacc.at[pl.ds(0, CH)], degsem).wait()

    plsc.subcore_barrier()

    sfill_start(0, 0)
    dfill_start(0, 0)
    sfill_wait(0)
    gstart(0)
    sfill_start(1, 1)
    dfill_start(1, 1)

    def body(j, carry):
        c = 2 * j
        sfill_wait(1)
        gstart(1)
        gwait(0)
        sfill_start(c + 2, 0)
        scatter(0)
        dfill_start(c + 2, 0)
        sfill_wait(0)
        gstart(0)
        gwait(1)

        @pl.when(c + 3 < NCHUNK)
        def _():
            sfill_start(c + 3, 1)

        scatter(1)

        @pl.when(c + 3 < NCHUNK)
        def _():
            dfill_start(c + 3, 1)

        return carry

    lax.fori_loop(0, (NCHUNK - 1) // 2, body, 0)
    gwait(0)
    scatter(0)

    plsc.subcore_barrier()

    @pl.when(cid == 0)
    def _():
        @pl.when(sid < 15)
        def _():
            st = pl.multiple_of(sid * ZR, 8)
            pltpu.sync_copy(acc.at[pl.ds(st, ZR)], out_lo.at[pl.ds(st, ZR)])
            pltpu.sync_copy(dacc.at[pl.ds(st, ZR)], dbuf.at[pl.ds(0, ZR)])
            pltpu.sync_copy(dbuf.at[pl.ds(0, ZR)], out_deg0.at[pl.ds(st, ZR)])

        @pl.when(sid == 15)
        def _():
            pltpu.sync_copy(acc.at[pl.ds(15 * ZR, ZR_LAST)],
                            out_lo.at[pl.ds(15 * ZR, ZR_LAST)])
            pltpu.sync_copy(dacc.at[pl.ds(15 * ZR, ZR_LAST)], dbuf)
            pltpu.sync_copy(dbuf, out_deg0.at[pl.ds(15 * ZR, ZR_LAST)])

    @pl.when(cid == 1)
    def _():
        @pl.when(sid < 15)
        def _():
            st = pl.multiple_of(sid * ZR, 8)
            pltpu.sync_copy(acc.at[pl.ds(st, ZR)], out_hi.at[pl.ds(st, ZR)])
            pltpu.sync_copy(dacc.at[pl.ds(st, ZR)], dbuf.at[pl.ds(0, ZR)])
            pltpu.sync_copy(dbuf.at[pl.ds(0, ZR)], out_deg1.at[pl.ds(st, ZR)])

        @pl.when(sid == 15)
        def _():
            pltpu.sync_copy(acc.at[pl.ds(15 * ZR, ZR_LAST)],
                            out_hi.at[pl.ds(15 * ZR, ZR_LAST)])
            pltpu.sync_copy(dacc.at[pl.ds(15 * ZR, ZR_LAST)], dbuf)
            pltpu.sync_copy(dbuf, out_deg1.at[pl.ds(15 * ZR, ZR_LAST)])


ROWS_BLK = 2000


def _tc_body(aglo_ref, aghi_ref, x_ref, dg0_ref, dg1_ref, w_ref, b_ref, o_ref):
    r = 1.0 / (dg0_ref[...] + dg1_ref[...] + 1.0)
    hlo = (aglo_ref[...] + x_ref[:, :DH]) * r
    hhi = (aghi_ref[...] + x_ref[:, DH:]) * r
    o_ref[...] = (
        jnp.dot(hlo, w_ref[:DH, :], preferred_element_type=jnp.float32)
        + jnp.dot(hhi, w_ref[DH:, :], preferred_element_type=jnp.float32)
        + b_ref[...]
    )


def _tc_call(agg_lo, agg_hi, x, deg0, deg1, W, b2):
    grid = (N // ROWS_BLK,)
    return pl.pallas_call(
        _tc_body,
        grid=grid,
        in_specs=[
            pl.BlockSpec((ROWS_BLK, DH), lambda i: (i, 0)),
            pl.BlockSpec((ROWS_BLK, DH), lambda i: (i, 0)),
            pl.BlockSpec((ROWS_BLK, D_IN), lambda i: (i, 0)),
            pl.BlockSpec((ROWS_BLK, 1), lambda i: (i, 0)),
            pl.BlockSpec((ROWS_BLK, 1), lambda i: (i, 0)),
            pl.BlockSpec((D_IN, D_IN), lambda i: (0, 0)),
            pl.BlockSpec((1, D_IN), lambda i: (0, 0)),
        ],
        out_specs=pl.BlockSpec((ROWS_BLK, D_IN), lambda i: (i, 0)),
        out_shape=jax.ShapeDtypeStruct((N, D_IN), jnp.float32),
    )(agg_lo, agg_hi, x, deg0, deg1, W, b2)


@jax.jit
def kernel(x, edge_index, W, b):
    src = edge_index[0].astype(jnp.int32)
    dst = edge_index[1].astype(jnp.int32)
    xs = x.reshape(2 * N, DH)
    src2 = jnp.concatenate([src * 2, src * 2 + 1])
    z128 = jnp.zeros((ZR_LAST, DH), jnp.float32)
    z1 = jnp.zeros((ZR_LAST,), jnp.float32)
    agg_lo, agg_hi, deg0, deg1 = _sc_agg(xs, src2, dst, z128, z1)
    return _tc_call(agg_lo, agg_hi, x, deg0.reshape(N, 1), deg1.reshape(N, 1),
                    W, b.reshape(1, D_IN))
